# Initial kernel scaffold; baseline (speedup 1.0000x reference)
import jax
import jax.numpy as jnp
from jax.experimental import pallas as pl


def kernel(fea, params):
    raise NotImplementedError("write your pallas kernel here")


# speedup vs baseline: 6.3197x; 6.3197x over previous
import functools

import jax
import jax.numpy as jnp
import numpy as np
from jax import lax
from jax.experimental import pallas as pl
from jax.experimental.pallas import tpu as pltpu
from jax.experimental.pallas import tpu_sc as plsc

KNN = 16
_BN_DIV = float(np.sqrt(np.float32(1.0 + 1e-5)))


def _lrelu(x):
    return jnp.where(x >= 0, x, 0.2 * x)


def _knn_body(xyzT_ref, xyzb_ref, nidx_ref, rd_ref, *, npts):
    b = pl.program_id(0)
    xT = xyzT_ref[0]
    xb = xyzb_ref[0]
    x2 = jnp.sum(xT * xT, axis=0, keepdims=True)
    x2b = jnp.sum(xb * xb, axis=1, keepdims=True)
    mm = jnp.dot(xb, xT, preferred_element_type=jnp.float32)
    d2 = jnp.maximum(x2b + x2 - 2.0 * mm, 1e-12)
    col = lax.broadcasted_iota(jnp.int32, d2.shape, 1)
    big = jnp.int32(0x7FFFFFFF)
    key = lax.bitcast_convert_type(d2, jnp.int32)
    vs, cs = [], []
    for _ in range(KNN):
        m = jnp.min(key, axis=1, keepdims=True)
        c = jnp.min(jnp.where(key == m, col, big), axis=1, keepdims=True)
        vs.append(m)
        cs.append(c)
        key = jnp.where(col == c, big, key)
    call = jnp.concatenate(cs, axis=1)
    mall = jnp.concatenate(vs, axis=1)
    nidx_ref[0] = call + b * npts
    rd_ref[0] = jnp.sqrt(lax.bitcast_convert_type(mall, jnp.float32))


def _knn(xyz, rows):
    B, N, _ = xyz.shape
    xyzT = jnp.swapaxes(xyz, 1, 2)
    grid = (B, N // rows)
    return pl.pallas_call(
        functools.partial(_knn_body, npts=N),
        grid=grid,
        in_specs=[
            pl.BlockSpec((1, 3, N), lambda b, i: (b, 0, 0)),
            pl.BlockSpec((1, rows, 3), lambda b, i: (b, i, 0)),
        ],
        out_specs=[
            pl.BlockSpec((1, rows, KNN), lambda b, i: (b, i, 0)),
            pl.BlockSpec((1, rows, KNN), lambda b, i: (b, i, 0)),
        ],
        out_shape=[
            jax.ShapeDtypeStruct((B, N, KNN), jnp.int32),
            jax.ShapeDtypeStruct((B, N, KNN), jnp.float32),
        ],
    )(xyzT, xyz)


def _sc_gather(table, idx2d, n_rows, n_cols):
    NW = 32
    CH = 128
    nch = n_rows // (NW * CH)
    mesh = plsc.VectorSubcoreMesh(core_axis_name="c", subcore_axis_name="s")

    @functools.partial(
        pl.kernel,
        out_type=jax.ShapeDtypeStruct((n_rows, n_cols), jnp.float32),
        mesh=mesh,
        scratch_types=[
            pltpu.VMEM((nch, CH), jnp.int32),
            pltpu.VMEM((CH, n_cols), jnp.float32),
            pltpu.SemaphoreType.DMA,
        ],
        compiler_params=pltpu.CompilerParams(use_tc_tiling_on_sc=False),
    )
    def g(table_hbm, idx_hbm, out_hbm, idx_v, rows_v, sem):
        wid = lax.axis_index("s") * 2 + lax.axis_index("c")
        pltpu.sync_copy(idx_hbm.at[pl.ds(wid * nch, nch)], idx_v)

        def body(j, carry):
            pltpu.async_copy(table_hbm.at[idx_v.at[j]], rows_v, sem).wait()
            pltpu.sync_copy(
                rows_v, out_hbm.at[pl.ds((wid * nch + j) * CH, CH)])
            return carry

        lax.fori_loop(0, nch, body, 0)

    return g(table, idx2d)


def _bn(y, g, be):
    return g * y / _BN_DIV + be


def _lfa_body(fg_ref, nx_ref, ex_ref, rd_ref, ori_ref,
              Wr, rb, rg, rbe, Wa, Wo, ob, og, obe, Wsc, sb, sg, sbe,
              *rest, P, in_c, rel_c, head):
    head_w = rest[:-1]
    out_ref = rest[-1]
    fg = fg_ref[0]
    nx = nx_ref[0]
    xb = ex_ref[0]
    rdb = rd_ref[0]
    ori = ori_ref[0]

    def bcast(v):
        c = v.shape[-1]
        return jnp.broadcast_to(
            v.reshape(P, 1, c), (P, KNN, c)).reshape(P * KNN, c)

    exn = bcast(xb)
    diff = exn - nx
    rowk = lax.broadcasted_iota(jnp.int32, (P * KNN, 16), 0) & (KNN - 1)
    lane = lax.broadcasted_iota(jnp.int32, (P * KNN, 16), 1)
    rd1 = jnp.sum(jnp.where(lane == rowk, bcast(rdb), 0.0),
                  axis=1, keepdims=True)
    rawrel = (jnp.where(lane == 0, rd1, 0.0)
              + jnp.roll(diff, 1, axis=1)
              + jnp.roll(exn, 4, axis=1)
              + jnp.roll(nx, 7, axis=1))

    rel = _lrelu(_bn(
        jnp.dot(rawrel, Wr[...], preferred_element_type=jnp.float32)
        + rb[...], rg[...], rbe[...]))

    f = jnp.concatenate([fg, rel], axis=1)
    logits = jnp.dot(f, Wa[...], preferred_element_type=jnp.float32)
    C = f.shape[-1]
    l3 = logits.reshape(P, KNN, C)
    m = jnp.max(l3, axis=1, keepdims=True)
    e = jnp.exp(l3 - m)

    def seqsum(x3):
        acc = x3[:, 0, :]
        for k in range(1, KNN):
            acc = acc + x3[:, k, :]
        return acc

    a = e / seqsum(e).reshape(P, 1, C)
    pooled = seqsum(a * f.reshape(P, KNN, C))

    ho = _bn(jnp.dot(pooled, Wo[...], preferred_element_type=jnp.float32)
             + ob[...], og[...], obe[...])
    hs = _bn(jnp.dot(ori, Wsc[...], preferred_element_type=jnp.float32)
             + sb[...], sg[...], sbe[...])
    out = _lrelu(hs + ho)

    if head is not None:
        W1, b1, g1, be1, W2, b2, g2, be2 = head_w
        t1 = _lrelu(_bn(
            jnp.dot(out, W1[...], preferred_element_type=jnp.float32)
            + b1[...], g1[...], be1[...]))
        t2 = _bn(jnp.dot(t1, W2[...], preferred_element_type=jnp.float32)
                 + b2[...], g2[...], be2[...])
        if head == 'enc':
            s = 1.0 / (1.0 + jnp.exp(-t2))
            out = jnp.where(s > 0.5, 1.0, 0.0)
        else:
            out = t2
    out_ref[0] = out


def _rowvecs(blk):
    n = blk['b'].shape[0]
    return [blk['b'].reshape(1, n), blk['g'].reshape(1, n),
            blk['be'].reshape(1, n)]


def _lfa_layer(fg, nxyz16, xyz16, rd, ori, p, P, head=None, head_p=None):
    B, N, in_c = ori.shape
    Wr = p['rel']['W']
    rel_c = Wr.shape[1]
    Wr16 = jnp.zeros((16, rel_c), jnp.float32).at[:10].set(Wr)
    Wa = p['W_attn']
    Wo = p['out']['W']
    Wsc = p['sc']['W']
    out_c = Wsc.shape[1]

    weights = ([Wr16] + _rowvecs(p['rel']) + [Wa, Wo] + _rowvecs(p['out'])
               + [Wsc] + _rowvecs(p['sc']))
    if head is not None:
        for hp in head_p:
            weights += [hp['W']] + _rowvecs(hp)
        out_c = head_p[1]['W'].shape[1]

    wspecs = [pl.BlockSpec(w.shape, lambda b, i, r=len(w.shape): (0,) * r)
              for w in weights]
    grid = (B, N // P)
    return pl.pallas_call(
        functools.partial(_lfa_body, P=P, in_c=in_c, rel_c=rel_c, head=head),
        grid=grid,
        in_specs=[
            pl.BlockSpec((1, P * KNN, in_c), lambda b, i: (b, i, 0)),
            pl.BlockSpec((1, P * KNN, 16), lambda b, i: (b, i, 0)),
            pl.BlockSpec((1, P, 16), lambda b, i: (b, i, 0)),
            pl.BlockSpec((1, P, KNN), lambda b, i: (b, i, 0)),
            pl.BlockSpec((1, P, in_c), lambda b, i: (b, i, 0)),
        ] + wspecs,
        out_specs=pl.BlockSpec((1, P, out_c), lambda b, i: (b, i, 0)),
        out_shape=jax.ShapeDtypeStruct((B, N, out_c), jnp.float32),
    )(fg, nxyz16, xyz16, rd, ori, *weights)


def _pad_layer1(p):
    ii = jnp.array([0, 1, 2] + list(range(16, 32)))
    p = dict(p)
    p['W_attn'] = jnp.zeros((32, 32), jnp.float32).at[
        ii[:, None], ii[None, :]].set(p['W_attn'])
    sc = dict(p['sc'])
    sc['W'] = jnp.zeros((16, sc['W'].shape[1]), jnp.float32).at[:3].set(
        sc['W'])
    p['sc'] = sc
    out = dict(p['out'])
    out['W'] = jnp.zeros((32, out['W'].shape[1]), jnp.float32).at[ii].set(
        out['W'])
    p['out'] = out
    return p


def kernel(fea, params):
    B, N, _ = fea.shape
    M = B * N * KNN
    P = 128
    xyz = fea

    nidx_g, rd = _knn(xyz, 128)
    idx2d = nidx_g.reshape(M // 128, 128)

    xyz16 = jnp.concatenate(
        [xyz.reshape(B * N, 3),
         jnp.zeros((B * N, 13), jnp.float32)], axis=1)
    nxyz16 = _sc_gather(xyz16, idx2d, M, 16)
    nxyz16 = nxyz16.reshape(B, N * KNN, 16)
    xyz16r = xyz16.reshape(B, N, 16)

    f = xyz16r
    fg = nxyz16
    enc = params['enc']
    for li, p in enumerate(enc):
        if li > 0:
            in_c = f.shape[-1]
            fg = _sc_gather(f.reshape(B * N, in_c), idx2d, M, in_c)
            fg = fg.reshape(B, N * KNN, in_c)
        else:
            p = _pad_layer1(p)
        head = 'enc' if li == len(enc) - 1 else None
        f = _lfa_layer(fg, nxyz16, xyz16r, rd, f, p, P, head=head,
                       head_p=params['enc_out'] if head else None)

    round_fea = f
    f2 = round_fea
    dec = params['dec']
    for li, p in enumerate(dec):
        in_c = f2.shape[-1]
        fg = _sc_gather(f2.reshape(B * N, in_c), idx2d, M, in_c)
        fg = fg.reshape(B, N * KNN, in_c)
        head = 'dec' if li == len(dec) - 1 else None
        f2 = _lfa_layer(fg, nxyz16, xyz16r, rd, f2, p, P, head=head,
                        head_p=params['dec_out'] if head else None)
    return round_fea, f2
